# baseline (speedup 1.0000x reference)
import functools
import math

import jax
import jax.numpy as jnp
from jax import lax
from jax.experimental import pallas as pl
from jax.experimental.pallas import tpu as pltpu
from jax.experimental.pallas import tpu_sc as plsc

_LOG_MIN = math.log(0.0 + 1e-08)
_LOG_MAX = math.log(1.0)

_NS = 16
_L = 16
_C0 = 416
_C1 = 608


def _make_gather_clip(batch):
    assert (_C0 + _C1) * _NS == batch

    @functools.partial(
        pl.kernel,
        out_type=jax.ShapeDtypeStruct((batch,), jnp.float32),
        mesh=plsc.VectorSubcoreMesh(core_axis_name="c", subcore_axis_name="s"),
        scratch_types=[
            pltpu.VMEM((_C0,), jnp.int32),
            pltpu.VMEM((_C0,), jnp.float32),
            pltpu.VMEM((_C1,), jnp.int32),
            pltpu.VMEM((_C1,), jnp.float32),
            pltpu.SemaphoreType.DMA,
            pltpu.SemaphoreType.DMA,
            pltpu.SemaphoreType.DMA,
        ],
    )
    def gather_clip(
        table_hbm, idx_hbm, out_hbm, idx0, vals0, idx1, vals1, g0sem, g1sem, osem
    ):
        cid = lax.axis_index("c")
        sid = lax.axis_index("s")

        def clamp(vals_v, lo, hi):
            for k in range(lo // _L, hi // _L):
                sl = pl.ds(k * _L, _L)
                v = vals_v[sl]
                vals_v[sl] = jnp.minimum(jnp.maximum(v, _LOG_MIN), _LOG_MAX)

        def run(base, idx_v, vals_v, count):
            h = count // 2
            pltpu.sync_copy(idx_hbm.at[pl.ds(base, count)], idx_v)
            g0 = pltpu.async_copy(
                table_hbm.at[idx_v.at[pl.ds(0, h)]], vals_v.at[pl.ds(0, h)], g0sem
            )
            g1 = pltpu.async_copy(
                table_hbm.at[idx_v.at[pl.ds(h, h)]], vals_v.at[pl.ds(h, h)], g1sem
            )
            g0.wait()
            clamp(vals_v, 0, h)
            o0 = pltpu.async_copy(
                vals_v.at[pl.ds(0, h)], out_hbm.at[pl.ds(base, h)], osem
            )
            g1.wait()
            clamp(vals_v, h, count)
            o1 = pltpu.async_copy(
                vals_v.at[pl.ds(h, h)], out_hbm.at[pl.ds(base + h, h)], osem
            )
            o0.wait()
            o1.wait()

        @pl.when(cid == 0)
        def _():
            run(sid * _C0, idx0, vals0, _C0)

        @pl.when(cid == 1)
        def _():
            run(_NS * _C0 + sid * _C1, idx1, vals1, _C1)

    return gather_clip


def kernel(log_coefs, coef_idxs):
    batch = coef_idxs.shape[0]
    out = _make_gather_clip(batch)(log_coefs, coef_idxs.astype(jnp.int32))
    return out.reshape(-1, 1)


# speedup vs baseline: 1.0038x; 1.0038x over previous
import functools
import math

import jax
import jax.numpy as jnp
from jax import lax
from jax.experimental import pallas as pl
from jax.experimental.pallas import tpu as pltpu
from jax.experimental.pallas import tpu_sc as plsc

_LOG_MIN = math.log(0.0 + 1e-08)
_LOG_MAX = math.log(1.0)

_NS = 16
_L = 16
_C0 = 352
_C1 = 672


def _make_gather_clip(batch):
    assert (_C0 + _C1) * _NS == batch

    @functools.partial(
        pl.kernel,
        out_type=jax.ShapeDtypeStruct((batch,), jnp.float32),
        mesh=plsc.VectorSubcoreMesh(core_axis_name="c", subcore_axis_name="s"),
        scratch_types=[
            pltpu.VMEM((_C0,), jnp.int32),
            pltpu.VMEM((_C0,), jnp.float32),
            pltpu.VMEM((_C1,), jnp.int32),
            pltpu.VMEM((_C1,), jnp.float32),
            pltpu.SemaphoreType.DMA,
            pltpu.SemaphoreType.DMA,
            pltpu.SemaphoreType.DMA,
        ],
    )
    def gather_clip(
        table_hbm, idx_hbm, out_hbm, idx0, vals0, idx1, vals1, g0sem, g1sem, osem
    ):
        cid = lax.axis_index("c")
        sid = lax.axis_index("s")

        def clamp(vals_v, lo, hi):
            for k in range(lo // _L, hi // _L):
                sl = pl.ds(k * _L, _L)
                v = vals_v[sl]
                vals_v[sl] = jnp.minimum(jnp.maximum(v, _LOG_MIN), _LOG_MAX)

        def run(base, idx_v, vals_v, count):
            h = count // 2
            pltpu.sync_copy(idx_hbm.at[pl.ds(base, count)], idx_v)
            g0 = pltpu.async_copy(
                table_hbm.at[idx_v.at[pl.ds(0, h)]], vals_v.at[pl.ds(0, h)], g0sem
            )
            g1 = pltpu.async_copy(
                table_hbm.at[idx_v.at[pl.ds(h, h)]], vals_v.at[pl.ds(h, h)], g1sem
            )
            g0.wait()
            clamp(vals_v, 0, h)
            o0 = pltpu.async_copy(
                vals_v.at[pl.ds(0, h)], out_hbm.at[pl.ds(base, h)], osem
            )
            g1.wait()
            clamp(vals_v, h, count)
            o1 = pltpu.async_copy(
                vals_v.at[pl.ds(h, h)], out_hbm.at[pl.ds(base + h, h)], osem
            )
            o0.wait()
            o1.wait()

        @pl.when(cid == 0)
        def _():
            run(sid * _C0, idx0, vals0, _C0)

        @pl.when(cid == 1)
        def _():
            run(_NS * _C0 + sid * _C1, idx1, vals1, _C1)

    return gather_clip


def kernel(log_coefs, coef_idxs):
    batch = coef_idxs.shape[0]
    out = _make_gather_clip(batch)(log_coefs, coef_idxs.astype(jnp.int32))
    return out.reshape(-1, 1)


# speedup vs baseline: 1.0078x; 1.0039x over previous
import functools
import math

import jax
import jax.numpy as jnp
from jax import lax
from jax.experimental import pallas as pl
from jax.experimental.pallas import tpu as pltpu
from jax.experimental.pallas import tpu_sc as plsc
from jax._src.pallas import mpmd as plmpmd

_LOG_MIN = math.log(0.0 + 1e-08)
_LOG_MAX = math.log(1.0)

_NC = 2
_NS = 16
_L = 16


def _make_gather_clip(batch):
    per_core = batch // _NC
    per_tile = per_core // _NS

    vec_mesh = plsc.VectorSubcoreMesh(core_axis_name="c", subcore_axis_name="s")
    scs_mesh = plsc.ScalarSubcoreMesh(axis_name="c")

    def scs_body(table_hbm, idx_hbm, out_hbm, idx_sh, idx_v, vals_v, rsem, dsem):
        del table_hbm, out_hbm, idx_v, vals_v, dsem
        cid = lax.axis_index("c")
        pltpu.sync_copy(idx_hbm.at[pl.ds(cid * per_core, per_core)], idx_sh)
        for s in range(_NS):
            pltpu.semaphore_signal(rsem, 1, device_id={"s": s})

    def tec_body(table_hbm, idx_hbm, out_hbm, idx_sh, idx_v, vals_v, rsem, dsem):
        del idx_hbm
        cid = lax.axis_index("c")
        sid = lax.axis_index("s")
        base = cid * per_core + sid * per_tile
        pltpu.semaphore_wait(rsem, 1)
        pltpu.sync_copy(idx_sh.at[pl.ds(sid * per_tile, per_tile)], idx_v)
        pltpu.async_copy(table_hbm.at[idx_v], vals_v, dsem).wait()
        for k in range(per_tile // _L):
            sl = pl.ds(k * _L, _L)
            v = vals_v[sl]
            vals_v[sl] = jnp.minimum(jnp.maximum(v, _LOG_MIN), _LOG_MAX)
        pltpu.sync_copy(vals_v, out_hbm.at[pl.ds(base, per_tile)])

    return plmpmd.mpmd_map(
        [(scs_mesh, scs_body), (vec_mesh, tec_body)],
        jax.ShapeDtypeStruct((batch,), jnp.float32),
        scratch_types=[
            (pltpu.VMEM_SHARED)((per_core,), jnp.int32),
            (pltpu.VMEM @ vec_mesh)((per_tile,), jnp.int32),
            (pltpu.VMEM @ vec_mesh)((per_tile,), jnp.float32),
            pltpu.SemaphoreType.REGULAR @ vec_mesh,
            pltpu.SemaphoreType.DMA @ vec_mesh,
        ],
    )


def kernel(log_coefs, coef_idxs):
    batch = coef_idxs.shape[0]
    out = _make_gather_clip(batch)(log_coefs, coef_idxs.astype(jnp.int32))
    return out.reshape(-1, 1)


# speedup vs baseline: 1.0214x; 1.0135x over previous
import functools
import math

import jax
import jax.numpy as jnp
from jax import lax
from jax.experimental import pallas as pl
from jax.experimental.pallas import tpu as pltpu
from jax.experimental.pallas import tpu_sc as plsc
from jax._src.pallas import mpmd as plmpmd

_LOG_MIN = math.log(0.0 + 1e-08)
_LOG_MAX = math.log(1.0)

_NS = 16
_L = 16
_T0 = 352
_T1 = 672


def _make_gather_clip(batch):
    assert (_T0 + _T1) * _NS == batch
    c0 = _T0 * _NS
    c1 = _T1 * _NS

    vec_mesh = plsc.VectorSubcoreMesh(core_axis_name="c", subcore_axis_name="s")
    scs_mesh = plsc.ScalarSubcoreMesh(axis_name="c")

    def scs_body(table_hbm, idx_hbm, out_hbm, idx_sh, idx_v, vals_v, rsem, dsem):
        del table_hbm, out_hbm, idx_v, vals_v, dsem
        cid = lax.axis_index("c")

        @pl.when(cid == 0)
        def _():
            pltpu.sync_copy(idx_hbm.at[pl.ds(0, c0)], idx_sh.at[pl.ds(0, c0)])

        @pl.when(cid == 1)
        def _():
            pltpu.sync_copy(idx_hbm.at[pl.ds(c0, c1)], idx_sh.at[pl.ds(0, c1)])

        for s in range(_NS):
            pltpu.semaphore_signal(rsem, 1, device_id={"s": s})

    def tec_body(table_hbm, idx_hbm, out_hbm, idx_sh, idx_v, vals_v, rsem, dsem):
        del idx_hbm
        cid = lax.axis_index("c")
        sid = lax.axis_index("s")
        pltpu.semaphore_wait(rsem, 1)

        def run(out_base, count):
            sl = pl.ds(0, count)
            pltpu.sync_copy(idx_sh.at[pl.ds(sid * count, count)], idx_v.at[sl])
            pltpu.async_copy(
                table_hbm.at[idx_v.at[sl]], vals_v.at[sl], dsem
            ).wait()
            for k in range(count // _L):
                ksl = pl.ds(k * _L, _L)
                v = vals_v[ksl]
                vals_v[ksl] = jnp.minimum(jnp.maximum(v, _LOG_MIN), _LOG_MAX)
            pltpu.sync_copy(vals_v.at[sl], out_hbm.at[pl.ds(out_base, count)])

        @pl.when(cid == 0)
        def _():
            run(sid * _T0, _T0)

        @pl.when(cid == 1)
        def _():
            run(c0 + sid * _T1, _T1)

    return plmpmd.mpmd_map(
        [(scs_mesh, scs_body), (vec_mesh, tec_body)],
        jax.ShapeDtypeStruct((batch,), jnp.float32),
        scratch_types=[
            (pltpu.VMEM_SHARED)((max(c0, c1),), jnp.int32),
            (pltpu.VMEM @ vec_mesh)((max(_T0, _T1),), jnp.int32),
            (pltpu.VMEM @ vec_mesh)((max(_T0, _T1),), jnp.float32),
            pltpu.SemaphoreType.REGULAR @ vec_mesh,
            pltpu.SemaphoreType.DMA @ vec_mesh,
        ],
    )


def kernel(log_coefs, coef_idxs):
    batch = coef_idxs.shape[0]
    out = _make_gather_clip(batch)(log_coefs, coef_idxs.astype(jnp.int32))
    return out.reshape(-1, 1)


# speedup vs baseline: 1.0241x; 1.0027x over previous
import functools
import math

import jax
import jax.numpy as jnp
from jax import lax
from jax.experimental import pallas as pl
from jax.experimental.pallas import tpu as pltpu
from jax.experimental.pallas import tpu_sc as plsc
from jax._src.pallas import mpmd as plmpmd

_LOG_MIN = math.log(0.0 + 1e-08)
_LOG_MAX = math.log(1.0)

_NS = 16
_L = 16
_T0 = 288
_T1 = 736


def _make_gather_clip(batch):
    assert (_T0 + _T1) * _NS == batch
    c0 = _T0 * _NS
    c1 = _T1 * _NS

    vec_mesh = plsc.VectorSubcoreMesh(core_axis_name="c", subcore_axis_name="s")
    scs_mesh = plsc.ScalarSubcoreMesh(axis_name="c")

    def scs_body(table_hbm, idx_hbm, out_hbm, idx_sh, idx_v, vals_v, rsem, dsem):
        del table_hbm, out_hbm, idx_v, vals_v, dsem
        cid = lax.axis_index("c")

        @pl.when(cid == 0)
        def _():
            pltpu.sync_copy(idx_hbm.at[pl.ds(0, c0)], idx_sh.at[pl.ds(0, c0)])

        @pl.when(cid == 1)
        def _():
            pltpu.sync_copy(idx_hbm.at[pl.ds(c0, c1)], idx_sh.at[pl.ds(0, c1)])

        for s in range(_NS):
            pltpu.semaphore_signal(rsem, 1, device_id={"s": s})

    def tec_body(table_hbm, idx_hbm, out_hbm, idx_sh, idx_v, vals_v, rsem, dsem):
        del idx_hbm
        cid = lax.axis_index("c")
        sid = lax.axis_index("s")
        pltpu.semaphore_wait(rsem, 1)

        def run(out_base, count):
            sl = pl.ds(0, count)
            pltpu.sync_copy(idx_sh.at[pl.ds(sid * count, count)], idx_v.at[sl])
            pltpu.async_copy(
                table_hbm.at[idx_v.at[sl]], vals_v.at[sl], dsem
            ).wait()
            for k in range(count // _L):
                ksl = pl.ds(k * _L, _L)
                v = vals_v[ksl]
                vals_v[ksl] = jnp.minimum(jnp.maximum(v, _LOG_MIN), _LOG_MAX)
            pltpu.sync_copy(vals_v.at[sl], out_hbm.at[pl.ds(out_base, count)])

        @pl.when(cid == 0)
        def _():
            run(sid * _T0, _T0)

        @pl.when(cid == 1)
        def _():
            run(c0 + sid * _T1, _T1)

    return plmpmd.mpmd_map(
        [(scs_mesh, scs_body), (vec_mesh, tec_body)],
        jax.ShapeDtypeStruct((batch,), jnp.float32),
        scratch_types=[
            (pltpu.VMEM_SHARED)((max(c0, c1),), jnp.int32),
            (pltpu.VMEM @ vec_mesh)((max(_T0, _T1),), jnp.int32),
            (pltpu.VMEM @ vec_mesh)((max(_T0, _T1),), jnp.float32),
            pltpu.SemaphoreType.REGULAR @ vec_mesh,
            pltpu.SemaphoreType.DMA @ vec_mesh,
        ],
    )


def kernel(log_coefs, coef_idxs):
    batch = coef_idxs.shape[0]
    out = _make_gather_clip(batch)(log_coefs, coef_idxs.astype(jnp.int32))
    return out.reshape(-1, 1)
